# Initial kernel scaffold; baseline (speedup 1.0000x reference)
import jax
import jax.numpy as jnp
from jax.experimental import pallas as pl


def kernel(feats, adjs, pm_W1, pm_b1, pm_W2, pm_b2, de_W1, de_b1, de_W2, de_b2, fg_W, fg_b, fusion_w, centers):
    raise NotImplementedError("write your pallas kernel here")


# speedup vs baseline: 2.5521x; 2.5521x over previous
import functools

import jax
import jax.numpy as jnp
from jax.experimental import pallas as pl

V = 3
N = 2000
D_IN = 256
H1 = 128
H2 = 64
K = 10
TM = 400
T = N // TM

_F32 = jnp.float32


def _softmax_w(fw_ref):
    e0 = jnp.exp(fw_ref[0, 0])
    e1 = jnp.exp(fw_ref[0, 1])
    e2 = jnp.exp(fw_ref[0, 2])
    s = e0 + e1 + e2
    return e0 / s, e1 / s, e2 / s


def _wv(fw_ref, v):
    w0, w1, w2 = _softmax_w(fw_ref)
    return jnp.where(v == 0, w0, jnp.where(v == 1, w1, w2))


def _proj1_kernel(x_ref, w1_ref, o_ref):
    o_ref[0] = jax.lax.dot_general(
        x_ref[0], w1_ref[0], (((1,), (0,)), ((), ())),
        preferred_element_type=_F32)


def _gcn1_kernel(a_ref, p1_ref, b1_ref, w2_ref, o_ref):
    h1 = jax.lax.dot_general(
        a_ref[0], p1_ref[0], (((1,), (0,)), ((), ())),
        preferred_element_type=_F32)
    h1 = jax.nn.relu(h1 + b1_ref[0])
    o_ref[0] = jax.lax.dot_general(
        h1, w2_ref[0], (((1,), (0,)), ((), ())),
        preferred_element_type=_F32)


def _gcn2_kernel(a_ref, p2_ref, b2_ref, dw1_ref, db1_ref, dw2_ref, db2_ref,
                 fgw_ref, fw_ref, h_ref, xb_ref, g_ref, cpr_ref, gsum_ref):
    v = pl.program_id(1)
    h = jax.lax.dot_general(
        a_ref[0], p2_ref[0], (((1,), (0,)), ((), ())),
        preferred_element_type=_F32)
    h = jax.nn.relu(h + b2_ref[0])
    h_ref[0] = h
    xb = jax.nn.relu(jax.lax.dot_general(
        h, dw1_ref[0], (((1,), (0,)), ((), ())),
        preferred_element_type=_F32) + db1_ref[0])
    xb_ref[0] = jax.nn.relu(jax.lax.dot_general(
        xb, dw2_ref[0], (((1,), (0,)), ((), ())),
        preferred_element_type=_F32) + db2_ref[0])
    g = jax.lax.dot_general(
        h, fgw_ref[...], (((1,), (0,)), ((), ())),
        preferred_element_type=_F32)
    g_ref[0] = g
    wv = _wv(fw_ref, v)

    @pl.when(v == 0)
    def _():
        cpr_ref[...] = wv * h
        gsum_ref[...] = g

    @pl.when(v > 0)
    def _():
        cpr_ref[...] += wv * h
        gsum_ref[...] += g


def _fg_kernel(ht_ref, hf_ref, gf_ref, gt_ref, gsum_ref, fgb_ref, fw_ref,
               cen_ref, adjbar_ref, comb_ref, q_ref):
    v = pl.program_id(1)
    s = jax.nn.sigmoid(jax.lax.dot_general(
        ht_ref[0], hf_ref[0], (((1,), (1,)), ((), ())),
        preferred_element_type=_F32))
    adjbar_ref[0] = s
    acc = jax.lax.dot_general(
        s, gf_ref[0], (((1,), (0,)), ((), ())),
        preferred_element_type=_F32)
    h_all = jax.nn.relu(acc + gsum_ref[...] - gt_ref[0] + fgb_ref[...])
    wv = _wv(fw_ref, v)

    @pl.when(v == 0)
    def _():
        comb_ref[...] = wv * h_all

    @pl.when(v > 0)
    def _():
        comb_ref[...] += wv * h_all

    @pl.when(v == V - 1)
    def _():
        c = comb_ref[...]
        diff = c[:, None, :] - cen_ref[...][None, :, :]
        dist = jnp.sum(diff * diff, axis=-1)
        q = 1.0 / (1.0 + dist)
        q_ref[...] = q / jnp.sum(q, axis=1, keepdims=True)


def kernel(feats, adjs, pm_W1, pm_b1, pm_W2, pm_b2, de_W1, de_b1, de_W2,
           de_b2, fg_W, fg_b, fusion_w, centers):
    f32 = _F32
    pm_b1r = pm_b1.reshape(V, 1, H1)
    pm_b2r = pm_b2.reshape(V, 1, H2)
    de_b1r = de_b1.reshape(V, 1, H1)
    de_b2r = de_b2.reshape(V, 1, D_IN)
    fg_br = fg_b.reshape(1, H2)
    fw = jnp.zeros((8, 128), f32).at[0, :V].set(fusion_w)

    p1 = pl.pallas_call(
        _proj1_kernel,
        grid=(V,),
        in_specs=[
            pl.BlockSpec((1, N, D_IN), lambda v: (v, 0, 0)),
            pl.BlockSpec((1, D_IN, H1), lambda v: (v, 0, 0)),
        ],
        out_specs=pl.BlockSpec((1, N, H1), lambda v: (v, 0, 0)),
        out_shape=jax.ShapeDtypeStruct((V, N, H1), f32),
    )(feats, pm_W1)

    p2 = pl.pallas_call(
        _gcn1_kernel,
        grid=(V, T),
        in_specs=[
            pl.BlockSpec((1, TM, N), lambda v, t: (v, t, 0)),
            pl.BlockSpec((1, N, H1), lambda v, t: (v, 0, 0)),
            pl.BlockSpec((1, 1, H1), lambda v, t: (v, 0, 0)),
            pl.BlockSpec((1, H1, H2), lambda v, t: (v, 0, 0)),
        ],
        out_specs=pl.BlockSpec((1, TM, H2), lambda v, t: (v, t, 0)),
        out_shape=jax.ShapeDtypeStruct((V, N, H2), f32),
    )(adjs, p1, pm_b1r, pm_W2)

    h, xbar, g, combined_pr, gsum = pl.pallas_call(
        _gcn2_kernel,
        grid=(T, V),
        in_specs=[
            pl.BlockSpec((1, TM, N), lambda t, v: (v, t, 0)),
            pl.BlockSpec((1, N, H2), lambda t, v: (v, 0, 0)),
            pl.BlockSpec((1, 1, H2), lambda t, v: (v, 0, 0)),
            pl.BlockSpec((1, H2, H1), lambda t, v: (v, 0, 0)),
            pl.BlockSpec((1, 1, H1), lambda t, v: (v, 0, 0)),
            pl.BlockSpec((1, H1, D_IN), lambda t, v: (v, 0, 0)),
            pl.BlockSpec((1, 1, D_IN), lambda t, v: (v, 0, 0)),
            pl.BlockSpec((H2, H2), lambda t, v: (0, 0)),
            pl.BlockSpec((8, 128), lambda t, v: (0, 0)),
        ],
        out_specs=[
            pl.BlockSpec((1, TM, H2), lambda t, v: (v, t, 0)),
            pl.BlockSpec((1, TM, D_IN), lambda t, v: (v, t, 0)),
            pl.BlockSpec((1, TM, H2), lambda t, v: (v, t, 0)),
            pl.BlockSpec((TM, H2), lambda t, v: (t, 0)),
            pl.BlockSpec((TM, H2), lambda t, v: (t, 0)),
        ],
        out_shape=[
            jax.ShapeDtypeStruct((V, N, H2), f32),
            jax.ShapeDtypeStruct((V, N, D_IN), f32),
            jax.ShapeDtypeStruct((V, N, H2), f32),
            jax.ShapeDtypeStruct((N, H2), f32),
            jax.ShapeDtypeStruct((N, H2), f32),
        ],
    )(adjs, p2, pm_b2r, de_W1, de_b1r, de_W2, de_b2r, fg_W, fw)

    adjbar, combined, q = pl.pallas_call(
        _fg_kernel,
        grid=(T, V),
        in_specs=[
            pl.BlockSpec((1, TM, H2), lambda t, v: (v, t, 0)),
            pl.BlockSpec((1, N, H2), lambda t, v: (v, 0, 0)),
            pl.BlockSpec((1, N, H2), lambda t, v: (v, 0, 0)),
            pl.BlockSpec((1, TM, H2), lambda t, v: (v, t, 0)),
            pl.BlockSpec((TM, H2), lambda t, v: (t, 0)),
            pl.BlockSpec((1, H2), lambda t, v: (0, 0)),
            pl.BlockSpec((8, 128), lambda t, v: (0, 0)),
            pl.BlockSpec((K, H2), lambda t, v: (0, 0)),
        ],
        out_specs=[
            pl.BlockSpec((1, TM, N), lambda t, v: (v, t, 0)),
            pl.BlockSpec((TM, H2), lambda t, v: (t, 0)),
            pl.BlockSpec((TM, K), lambda t, v: (t, 0)),
        ],
        out_shape=[
            jax.ShapeDtypeStruct((V, N, N), f32),
            jax.ShapeDtypeStruct((N, H2), f32),
            jax.ShapeDtypeStruct((N, K), f32),
        ],
    )(h, h, g, g, gsum, fg_br, fw, centers)

    return (combined, combined_pr, q, xbar, adjbar)


# speedup vs baseline: 3.0322x; 1.1881x over previous
import functools

import jax
import jax.numpy as jnp
from jax.experimental import pallas as pl

V = 3
N = 2000
D_IN = 256
H1 = 128
H2 = 64
K = 10
TM = 400
T = N // TM

_F32 = jnp.float32


def _softmax_w(fw_ref):
    e0 = jnp.exp(fw_ref[0, 0])
    e1 = jnp.exp(fw_ref[0, 1])
    e2 = jnp.exp(fw_ref[0, 2])
    s = e0 + e1 + e2
    return e0 / s, e1 / s, e2 / s


def _wv(fw_ref, v):
    w0, w1, w2 = _softmax_w(fw_ref)
    return jnp.where(v == 0, w0, jnp.where(v == 1, w1, w2))


def _gcn_kernel(x_ref, a_ref, w1_ref, b1_ref, w2_ref, b2_ref,
                dw1_ref, db1_ref, dw2_ref, db2_ref,
                fgw_ref, fw_ref, h_ref, xb_ref, g_ref, cpr_ref, gsum_ref):
    v = pl.program_id(0)
    a = a_ref[0]
    p1 = jax.lax.dot_general(
        x_ref[0], w1_ref[0], (((1,), (0,)), ((), ())),
        preferred_element_type=_F32)
    h1 = jax.nn.relu(jax.lax.dot_general(
        a, p1, (((1,), (0,)), ((), ())),
        preferred_element_type=_F32) + b1_ref[0])
    p2 = jax.lax.dot_general(
        h1, w2_ref[0], (((1,), (0,)), ((), ())),
        preferred_element_type=_F32)
    h = jax.nn.relu(jax.lax.dot_general(
        a, p2, (((1,), (0,)), ((), ())),
        preferred_element_type=_F32) + b2_ref[0])
    h_ref[0] = h
    xb = jax.nn.relu(jax.lax.dot_general(
        h, dw1_ref[0], (((1,), (0,)), ((), ())),
        preferred_element_type=_F32) + db1_ref[0])
    xb_ref[0] = jax.nn.relu(jax.lax.dot_general(
        xb, dw2_ref[0], (((1,), (0,)), ((), ())),
        preferred_element_type=_F32) + db2_ref[0])
    g = jax.lax.dot_general(
        h, fgw_ref[...], (((1,), (0,)), ((), ())),
        preferred_element_type=_F32)
    g_ref[0] = g
    wv = _wv(fw_ref, v)

    @pl.when(v == 0)
    def _():
        cpr_ref[...] = wv * h
        gsum_ref[...] = g

    @pl.when(v > 0)
    def _():
        cpr_ref[...] += wv * h
        gsum_ref[...] += g


def _fg_kernel(ht_ref, hf_ref, gf_ref, gt_ref, gsum_ref, fgb_ref, fw_ref,
               cen_ref, adjbar_ref, comb_ref, q_ref):
    v = pl.program_id(1)
    s = jax.nn.sigmoid(jax.lax.dot_general(
        ht_ref[0], hf_ref[0], (((1,), (1,)), ((), ())),
        preferred_element_type=_F32))
    adjbar_ref[0] = s
    acc = jax.lax.dot_general(
        s, gf_ref[0], (((1,), (0,)), ((), ())),
        preferred_element_type=_F32)
    h_all = jax.nn.relu(acc + gsum_ref[...] - gt_ref[0] + fgb_ref[...])
    wv = _wv(fw_ref, v)

    @pl.when(v == 0)
    def _():
        comb_ref[...] = wv * h_all

    @pl.when(v > 0)
    def _():
        comb_ref[...] += wv * h_all

    @pl.when(v == V - 1)
    def _():
        c = comb_ref[...]
        diff = c[:, None, :] - cen_ref[...][None, :, :]
        dist = jnp.sum(diff * diff, axis=-1)
        q = 1.0 / (1.0 + dist)
        q_ref[...] = q / jnp.sum(q, axis=1, keepdims=True)


def kernel(feats, adjs, pm_W1, pm_b1, pm_W2, pm_b2, de_W1, de_b1, de_W2,
           de_b2, fg_W, fg_b, fusion_w, centers):
    f32 = _F32
    pm_b1r = pm_b1.reshape(V, 1, H1)
    pm_b2r = pm_b2.reshape(V, 1, H2)
    de_b1r = de_b1.reshape(V, 1, H1)
    de_b2r = de_b2.reshape(V, 1, D_IN)
    fg_br = fg_b.reshape(1, H2)
    fw = jnp.zeros((8, 128), f32).at[0, :V].set(fusion_w)

    h, xbar, g, combined_pr, gsum = pl.pallas_call(
        _gcn_kernel,
        grid=(V,),
        in_specs=[
            pl.BlockSpec((1, N, D_IN), lambda v: (v, 0, 0)),
            pl.BlockSpec((1, N, N), lambda v: (v, 0, 0)),
            pl.BlockSpec((1, D_IN, H1), lambda v: (v, 0, 0)),
            pl.BlockSpec((1, 1, H1), lambda v: (v, 0, 0)),
            pl.BlockSpec((1, H1, H2), lambda v: (v, 0, 0)),
            pl.BlockSpec((1, 1, H2), lambda v: (v, 0, 0)),
            pl.BlockSpec((1, H2, H1), lambda v: (v, 0, 0)),
            pl.BlockSpec((1, 1, H1), lambda v: (v, 0, 0)),
            pl.BlockSpec((1, H1, D_IN), lambda v: (v, 0, 0)),
            pl.BlockSpec((1, 1, D_IN), lambda v: (v, 0, 0)),
            pl.BlockSpec((H2, H2), lambda v: (0, 0)),
            pl.BlockSpec((8, 128), lambda v: (0, 0)),
        ],
        out_specs=[
            pl.BlockSpec((1, N, H2), lambda v: (v, 0, 0)),
            pl.BlockSpec((1, N, D_IN), lambda v: (v, 0, 0)),
            pl.BlockSpec((1, N, H2), lambda v: (v, 0, 0)),
            pl.BlockSpec((N, H2), lambda v: (0, 0)),
            pl.BlockSpec((N, H2), lambda v: (0, 0)),
        ],
        out_shape=[
            jax.ShapeDtypeStruct((V, N, H2), f32),
            jax.ShapeDtypeStruct((V, N, D_IN), f32),
            jax.ShapeDtypeStruct((V, N, H2), f32),
            jax.ShapeDtypeStruct((N, H2), f32),
            jax.ShapeDtypeStruct((N, H2), f32),
        ],
    )(feats, adjs, pm_W1, pm_b1r, pm_W2, pm_b2r,
      de_W1, de_b1r, de_W2, de_b2r, fg_W, fw)

    adjbar, combined, q = pl.pallas_call(
        _fg_kernel,
        grid=(T, V),
        in_specs=[
            pl.BlockSpec((1, TM, H2), lambda t, v: (v, t, 0)),
            pl.BlockSpec((1, N, H2), lambda t, v: (v, 0, 0)),
            pl.BlockSpec((1, N, H2), lambda t, v: (v, 0, 0)),
            pl.BlockSpec((1, TM, H2), lambda t, v: (v, t, 0)),
            pl.BlockSpec((TM, H2), lambda t, v: (t, 0)),
            pl.BlockSpec((1, H2), lambda t, v: (0, 0)),
            pl.BlockSpec((8, 128), lambda t, v: (0, 0)),
            pl.BlockSpec((K, H2), lambda t, v: (0, 0)),
        ],
        out_specs=[
            pl.BlockSpec((1, TM, N), lambda t, v: (v, t, 0)),
            pl.BlockSpec((TM, H2), lambda t, v: (t, 0)),
            pl.BlockSpec((TM, K), lambda t, v: (t, 0)),
        ],
        out_shape=[
            jax.ShapeDtypeStruct((V, N, N), f32),
            jax.ShapeDtypeStruct((N, H2), f32),
            jax.ShapeDtypeStruct((N, K), f32),
        ],
    )(h, h, g, g, gsum, fg_br, fw, centers)

    return (combined, combined_pr, q, xbar, adjbar)
